# Initial kernel scaffold; baseline (speedup 1.0000x reference)
import jax
import jax.numpy as jnp
from jax.experimental import pallas as pl


def kernel(inputs, tables):
    raise NotImplementedError("write your pallas kernel here")


# speedup vs baseline: 102.2615x; 102.2615x over previous
import functools

import jax
import jax.numpy as jnp
from jax import lax
from jax.experimental import pallas as pl
from jax.experimental.pallas import tpu as pltpu
from jax.experimental.pallas import tpu_sc as plsc

_NT = 10
_E = 3
_V = 1_000_000
_B = 16384
_L = 200
_DP = 32
_NC, _NS = 2, 16
_NW = _NC * _NS
_BW = _B // _NW
_CH = 128
_NCH = _BW // _CH


def _sc_body(idx_hbm, tab_hbm, out_hbm, idx_v, rows_v, res_v, s0, s1, s2, s3):
    sems = (s0, s1, s2, s3)
    wid = lax.axis_index("s") * _NC + lax.axis_index("c")
    base = wid * _BW

    pltpu.sync_copy(idx_hbm.at[:, pl.ds(base, _BW)], idx_v)

    def issue(l, k):
        pltpu.async_copy(
            tab_hbm.at[idx_v.at[l, pl.ds(k * _CH, _CH)]], rows_v.at[k], sems[k]
        )

    issue(0, 0)
    issue(0, 1)

    def l_body(l, carry):
        z = jnp.zeros((16,), jnp.float32)
        a0, a1 = z, z
        for k in range(_NCH):
            if k < 2:
                issue(l, k + 2)
            else:

                @pl.when(l + 1 < _L)
                def _():
                    issue(jnp.minimum(l + 1, _L - 1), k - 2)

            pltpu.make_async_copy(
                tab_hbm.at[pl.ds(0, _CH)], rows_v.at[k], sems[k]
            ).wait()

            def r_body(r8, c):
                b0, b1 = c
                for j in range(8):
                    r = r8 * 8 + j
                    b0 = b0 + rows_v[k, r, 0:16]
                    b1 = b1 + rows_v[k, r, 16:32]
                return b0, b1

            a0, a1 = lax.fori_loop(0, _CH // 8, r_body, (a0, a1))
        res_v[l, 0:16] = a0
        res_v[l, 16:32] = a1
        return carry

    lax.fori_loop(0, _L, l_body, 0)
    pltpu.sync_copy(res_v, out_hbm.at[wid])


_gather_sum = functools.partial(
    pl.kernel,
    out_type=jax.ShapeDtypeStruct((_NW, _L, _DP), jnp.float32),
    mesh=plsc.VectorSubcoreMesh(core_axis_name="c", subcore_axis_name="s"),
    compiler_params=pltpu.CompilerParams(use_tc_tiling_on_sc=False),
    scratch_types=[
        pltpu.VMEM((_L, _BW), jnp.int32),
        pltpu.VMEM((_NCH, _CH, _DP), jnp.float32),
        pltpu.VMEM((_L, _DP), jnp.float32),
        pltpu.SemaphoreType.DMA,
        pltpu.SemaphoreType.DMA,
        pltpu.SemaphoreType.DMA,
        pltpu.SemaphoreType.DMA,
    ],
)(_sc_body)


def kernel(inputs, tables):
    t2 = tables.reshape(_NT, _V * _E).T.reshape(_V, _E * _NT)
    tabp = jnp.pad(t2, ((0, 0), (0, _DP - _E * _NT)))
    idx_t = inputs.T
    parts = _gather_sum(idx_t, tabp)
    m = jnp.sum(parts, axis=0)[:, : _E * _NT].reshape(_L, _E, _NT)
    m = jnp.transpose(m, (0, 2, 1))
    s3 = jnp.sum(m[:, 3, :], axis=0, keepdims=True)
    return jnp.concatenate(
        [m[:, 0], m[:, 1], m[:, 2], s3, m[:, 4], s3,
         m[:, 6], m[:, 7], m[:, 8], m[:, 9]],
        axis=0,
    )


# speedup vs baseline: 152.5416x; 1.4917x over previous
import functools

import jax
import jax.numpy as jnp
from jax import lax
from jax.experimental import pallas as pl
from jax.experimental.pallas import tpu as pltpu
from jax.experimental.pallas import tpu_sc as plsc

_NT = 10
_E = 3
_V = 1_000_000
_B = 16384
_L = 200
_D = _NT * _E
_DP = 32
_NC, _NS = 2, 16
_NW = _NC * _NS
_BW = _B // _NW
_CH = 128
_NCH = _BW // _CH

_TC = 800
_NG = _V // _TC
_GI = (_NG + _NW - 1) // _NW


def _relayout_body(tab30_hbm, out_hbm, in_v, out_v, si0, si1):
    sems = (si0, si1)
    wid = lax.axis_index("s") * _NC + lax.axis_index("c")
    lanes = lax.iota(jnp.int32, 16) * _DP

    def issue(gi, b):
        g = wid + _NW * gi
        pltpu.async_copy(
            tab30_hbm.at[:, pl.ds(g * _TC, _TC)], in_v.at[b], sems[b]
        )

    issue(0, 0)
    issue(1, 1)

    def slot_body(gi, b):
        g = wid + _NW * gi

        @pl.when(g < _NG)
        def _():
            pltpu.make_async_copy(
                tab30_hbm.at[:, pl.ds(0, _TC)], in_v.at[b], sems[b]
            ).wait()

            def r_body(r, carry):
                col = lanes + r
                for k in range(_TC // 16):
                    vals = in_v[b, r, k * 16:(k + 1) * 16]
                    plsc.store_scatter(
                        out_v.at[b], [col + (k * 16 * _DP)], vals
                    )
                return carry

            lax.fori_loop(0, _D, r_body, 0)

            @pl.when(g + 2 * _NW < _NG)
            def _():
                issue(gi + 2, b)

            pltpu.sync_copy(
                out_v.at[b], out_hbm.at[pl.ds(g * _TC * _DP, _TC * _DP)]
            )

    def it_body(it, carry):
        slot_body(2 * it, 0)
        slot_body(2 * it + 1, 1)
        return carry

    lax.fori_loop(0, _GI // 2, it_body, 0)


_relayout = functools.partial(
    pl.kernel,
    out_type=jax.ShapeDtypeStruct((_V * _DP,), jnp.float32),
    mesh=plsc.VectorSubcoreMesh(core_axis_name="c", subcore_axis_name="s"),
    compiler_params=pltpu.CompilerParams(
        use_tc_tiling_on_sc=False, needs_layout_passes=False
    ),
    scratch_types=[
        pltpu.VMEM((2, _D, _TC), jnp.float32),
        pltpu.VMEM((2, _TC * _DP), jnp.float32),
        pltpu.SemaphoreType.DMA,
        pltpu.SemaphoreType.DMA,
    ],
)(_relayout_body)


def _gather_body(idx_hbm, tab_hbm, out_hbm, idx_v, rows_v, res_v, s0, s1, s2, s3):
    sems = (s0, s1, s2, s3)
    wid = lax.axis_index("s") * _NC + lax.axis_index("c")
    base = wid * _BW

    pltpu.sync_copy(idx_hbm.at[:, pl.ds(base, _BW)], idx_v)

    def issue(l, k):
        pltpu.async_copy(
            tab_hbm.at[idx_v.at[l, pl.ds(k * _CH, _CH)]], rows_v.at[k], sems[k]
        )

    issue(0, 0)
    issue(0, 1)

    def l_body(l, carry):
        z = jnp.zeros((16,), jnp.float32)
        a0, a1 = z, z
        for k in range(_NCH):
            if k < 2:
                issue(l, k + 2)
            else:

                @pl.when(l + 1 < _L)
                def _():
                    issue(jnp.minimum(l + 1, _L - 1), k - 2)

            pltpu.make_async_copy(
                tab_hbm.at[pl.ds(0, _CH)], rows_v.at[k], sems[k]
            ).wait()

            def r_body(r8, c):
                b0, b1 = c
                for j in range(8):
                    r = r8 * 8 + j
                    b0 = b0 + rows_v[k, r, 0:16]
                    b1 = b1 + rows_v[k, r, 16:32]
                return b0, b1

            a0, a1 = lax.fori_loop(0, _CH // 8, r_body, (a0, a1))
        res_v[l, 0:16] = a0
        res_v[l, 16:32] = a1
        return carry

    lax.fori_loop(0, _L, l_body, 0)
    pltpu.sync_copy(res_v, out_hbm.at[wid])


_gather_sum = functools.partial(
    pl.kernel,
    out_type=jax.ShapeDtypeStruct((_NW, _L, _DP), jnp.float32),
    mesh=plsc.VectorSubcoreMesh(core_axis_name="c", subcore_axis_name="s"),
    compiler_params=pltpu.CompilerParams(use_tc_tiling_on_sc=False),
    scratch_types=[
        pltpu.VMEM((_L, _BW), jnp.int32),
        pltpu.VMEM((_NCH, _CH, _DP), jnp.float32),
        pltpu.VMEM((_L, _DP), jnp.float32),
        pltpu.SemaphoreType.DMA,
        pltpu.SemaphoreType.DMA,
        pltpu.SemaphoreType.DMA,
        pltpu.SemaphoreType.DMA,
    ],
)(_gather_body)


def kernel(inputs, tables):
    tab30 = jnp.transpose(tables, (0, 2, 1)).reshape(_D, _V)
    tabp = _relayout(tab30).reshape(_V, _DP)
    idx_t = inputs.T
    parts = _gather_sum(idx_t, tabp)
    m = jnp.sum(parts, axis=0)[:, :_D].reshape(_L, _NT, _E)
    s3 = jnp.sum(m[:, 3, :], axis=0, keepdims=True)
    return jnp.concatenate(
        [m[:, 0], m[:, 1], m[:, 2], s3, m[:, 4], s3,
         m[:, 6], m[:, 7], m[:, 8], m[:, 9]],
        axis=0,
    )


# speedup vs baseline: 342.2137x; 2.2434x over previous
import functools

import jax
import jax.numpy as jnp
from jax import lax
from jax.experimental import pallas as pl
from jax.experimental.pallas import tpu as pltpu
from jax.experimental.pallas import tpu_sc as plsc

_NT = 10
_E = 3
_V = 1_000_000
_B = 16384
_L = 200
_D = _NT * _E
_DP = 32
_NC, _NS = 2, 16
_NW = _NC * _NS
_BW = _B // _NW
_CH = 128
_NCH = _BW // _CH

_TC = 800
_NG = _V // _TC
_GI = (_NG + _NW - 1) // _NW


def _relayout_body(*refs):
    planes = refs[:_D]
    out_hbm = refs[_D]
    in_v, out_v, si0, si1 = refs[_D + 1:]
    sems = (si0, si1)
    wid = lax.axis_index("s") * _NC + lax.axis_index("c")
    lanes = lax.iota(jnp.int32, 16) * _DP

    def issue(gi, b):
        g = wid + _NW * gi
        for r in range(_D):
            pltpu.async_copy(
                planes[r].at[pl.ds(g * _TC, _TC)], in_v.at[b, r], sems[b]
            )

    issue(0, 0)
    issue(1, 1)

    def slot_body(gi, b):
        g = wid + _NW * gi

        @pl.when(g < _NG)
        def _():
            for r in range(_D):
                pltpu.make_async_copy(
                    planes[0].at[pl.ds(0, _TC)], in_v.at[b, r], sems[b]
                ).wait()

            def r_body(r, carry):
                col = lanes + r
                for k in range(_TC // 16):
                    vals = in_v[b, r, k * 16:(k + 1) * 16]
                    plsc.store_scatter(
                        out_v.at[b], [col + (k * 16 * _DP)], vals
                    )
                return carry

            lax.fori_loop(0, _D, r_body, 0)

            @pl.when(g + 2 * _NW < _NG)
            def _():
                issue(gi + 2, b)

            pltpu.sync_copy(
                out_v.at[b], out_hbm.at[pl.ds(g * _TC * _DP, _TC * _DP)]
            )

    def it_body(it, carry):
        slot_body(2 * it, 0)
        slot_body(2 * it + 1, 1)
        return carry

    lax.fori_loop(0, _GI // 2, it_body, 0)


_relayout = functools.partial(
    pl.kernel,
    out_type=jax.ShapeDtypeStruct((_V * _DP,), jnp.float32),
    mesh=plsc.VectorSubcoreMesh(core_axis_name="c", subcore_axis_name="s"),
    compiler_params=pltpu.CompilerParams(
        use_tc_tiling_on_sc=False, needs_layout_passes=False
    ),
    scratch_types=[
        pltpu.VMEM((2, _D, _TC), jnp.float32),
        pltpu.VMEM((2, _TC * _DP), jnp.float32),
        pltpu.SemaphoreType.DMA,
        pltpu.SemaphoreType.DMA,
    ],
)(_relayout_body)


def _gather_body(idx_hbm, tab_hbm, out_hbm, idx_v, rows_v, res_v, s0, s1, s2, s3):
    sems = (s0, s1, s2, s3)
    wid = lax.axis_index("s") * _NC + lax.axis_index("c")
    base = wid * _BW

    pltpu.sync_copy(idx_hbm.at[:, pl.ds(base, _BW)], idx_v)

    def issue(l, k):
        pltpu.async_copy(
            tab_hbm.at[idx_v.at[l, pl.ds(k * _CH, _CH)]], rows_v.at[k], sems[k]
        )

    issue(0, 0)
    issue(0, 1)

    def l_body(l, carry):
        z = jnp.zeros((16,), jnp.float32)
        a0, a1 = z, z
        for k in range(_NCH):
            if k < 2:
                issue(l, k + 2)
            else:

                @pl.when(l + 1 < _L)
                def _():
                    issue(jnp.minimum(l + 1, _L - 1), k - 2)

            pltpu.make_async_copy(
                tab_hbm.at[pl.ds(0, _CH)], rows_v.at[k], sems[k]
            ).wait()

            def r_body(r8, c):
                b0, b1 = c
                for j in range(8):
                    r = r8 * 8 + j
                    b0 = b0 + rows_v[k, r, 0:16]
                    b1 = b1 + rows_v[k, r, 16:32]
                return b0, b1

            a0, a1 = lax.fori_loop(0, _CH // 8, r_body, (a0, a1))
        res_v[l, 0:16] = a0
        res_v[l, 16:32] = a1
        return carry

    lax.fori_loop(0, _L, l_body, 0)
    pltpu.sync_copy(res_v, out_hbm.at[wid])


_gather_sum = functools.partial(
    pl.kernel,
    out_type=jax.ShapeDtypeStruct((_NW, _L, _DP), jnp.float32),
    mesh=plsc.VectorSubcoreMesh(core_axis_name="c", subcore_axis_name="s"),
    compiler_params=pltpu.CompilerParams(use_tc_tiling_on_sc=False),
    scratch_types=[
        pltpu.VMEM((_L, _BW), jnp.int32),
        pltpu.VMEM((_NCH, _CH, _DP), jnp.float32),
        pltpu.VMEM((_L, _DP), jnp.float32),
        pltpu.SemaphoreType.DMA,
        pltpu.SemaphoreType.DMA,
        pltpu.SemaphoreType.DMA,
        pltpu.SemaphoreType.DMA,
    ],
)(_gather_body)


def kernel(inputs, tables):
    planes = [tables[t, :, c] for t in range(_NT) for c in range(_E)]
    tabp = _relayout(*planes).reshape(_V, _DP)
    idx_t = inputs.T
    parts = _gather_sum(idx_t, tabp)
    m = jnp.sum(parts, axis=0)[:, :_D].reshape(_L, _NT, _E)
    s3 = jnp.sum(m[:, 3, :], axis=0, keepdims=True)
    return jnp.concatenate(
        [m[:, 0], m[:, 1], m[:, 2], s3, m[:, 4], s3,
         m[:, 6], m[:, 7], m[:, 8], m[:, 9]],
        axis=0,
    )


# speedup vs baseline: 450.7800x; 1.3172x over previous
import functools

import jax
import jax.numpy as jnp
from jax import lax
from jax.experimental import pallas as pl
from jax.experimental.pallas import tpu as pltpu
from jax.experimental.pallas import tpu_sc as plsc

_NT = 10
_E = 3
_V = 1_000_000
_B = 16384
_L = 200
_D = _NT * _E
_DP = 32
_NC, _NS = 2, 16
_NW = _NC * _NS
_BW = _B // _NW
_CH = 128
_NCH = _BW // _CH

_TC = 800
_TCP = 817
_NG = _V // _TC
_GI = (_NG + _NW - 1) // _NW


def _relayout_body(*refs):
    planes = refs[:_D]
    out_hbm = refs[_D]
    in_v, out_v, si0, si1, so0, so1 = refs[_D + 1:]
    sems = (si0, si1)
    osems = (so0, so1)
    wid = lax.axis_index("s") * _NC + lax.axis_index("c")
    lane = lax.iota(jnp.int32, 16)

    def issue(gi, b):
        g = wid + _NW * gi
        for r in range(_D):
            pltpu.async_copy(
                planes[r].at[pl.ds(g * _TC, _TC)],
                in_v.at[b, r, pl.ds(0, _TC)], sems[b]
            )

    issue(0, 0)
    issue(1, 1)

    def slot_body(gi, b):
        g = wid + _NW * gi

        @pl.when(g < _NG)
        def _():
            for r in range(_D):
                pltpu.make_async_copy(
                    planes[0].at[pl.ds(0, _TC)],
                    in_v.at[b, r, pl.ds(0, _TC)], sems[b]
                ).wait()

            @pl.when(gi >= 2)
            def _():
                pltpu.make_async_copy(
                    out_v.at[b], out_hbm.at[pl.ds(0, _TC)], osems[b]
                ).wait()

            def i_body(i4, carry):
                for u in range(4):
                    i = i4 * 4 + u
                    ci = jnp.full((16,), i, jnp.int32)
                    v0 = plsc.load_gather(in_v.at[b], [lane, ci])
                    v1 = plsc.load_gather(
                        in_v.at[b], [jnp.minimum(lane + 16, _D - 1), ci]
                    )
                    out_v[b, i, 0:16] = v0
                    out_v[b, i, 16:32] = v1
                return carry

            lax.fori_loop(0, _TC // 4, i_body, 0)

            @pl.when(g + 2 * _NW < _NG)
            def _():
                issue(gi + 2, b)

            pltpu.async_copy(
                out_v.at[b], out_hbm.at[pl.ds(g * _TC, _TC)], osems[b]
            )

    def it_body(it, carry):
        slot_body(2 * it, 0)
        slot_body(2 * it + 1, 1)
        return carry

    lax.fori_loop(0, _GI // 2, it_body, 0)
    for b in range(2):
        pltpu.make_async_copy(
            out_v.at[b], out_hbm.at[pl.ds(0, _TC)], osems[b]
        ).wait()


_relayout = functools.partial(
    pl.kernel,
    out_type=jax.ShapeDtypeStruct((_V, _DP), jnp.float32),
    mesh=plsc.VectorSubcoreMesh(core_axis_name="c", subcore_axis_name="s"),
    compiler_params=pltpu.CompilerParams(
        use_tc_tiling_on_sc=False, needs_layout_passes=False
    ),
    scratch_types=[
        pltpu.VMEM((2, _D, _TCP), jnp.float32),
        pltpu.VMEM((2, _TC, _DP), jnp.float32),
        pltpu.SemaphoreType.DMA,
        pltpu.SemaphoreType.DMA,
        pltpu.SemaphoreType.DMA,
        pltpu.SemaphoreType.DMA,
    ],
)(_relayout_body)


def _gather_body(idx_hbm, tab_hbm, out_hbm, idx_v, rows_v, res_v, s0, s1, s2, s3):
    sems = (s0, s1, s2, s3)
    wid = lax.axis_index("s") * _NC + lax.axis_index("c")
    base = wid * _BW

    pltpu.sync_copy(idx_hbm.at[:, pl.ds(base, _BW)], idx_v)

    def issue(l, k):
        pltpu.async_copy(
            tab_hbm.at[idx_v.at[l, pl.ds(k * _CH, _CH)]], rows_v.at[k], sems[k]
        )

    issue(0, 0)
    issue(0, 1)

    def l_body(l, carry):
        z = jnp.zeros((16,), jnp.float32)
        a0, a1 = z, z
        for k in range(_NCH):
            if k < 2:
                issue(l, k + 2)
            else:

                @pl.when(l + 1 < _L)
                def _():
                    issue(jnp.minimum(l + 1, _L - 1), k - 2)

            pltpu.make_async_copy(
                tab_hbm.at[pl.ds(0, _CH)], rows_v.at[k], sems[k]
            ).wait()

            def r_body(r8, c):
                b0, b1 = c
                for j in range(8):
                    r = r8 * 8 + j
                    b0 = b0 + rows_v[k, r, 0:16]
                    b1 = b1 + rows_v[k, r, 16:32]
                return b0, b1

            a0, a1 = lax.fori_loop(0, _CH // 8, r_body, (a0, a1))
        res_v[l, 0:16] = a0
        res_v[l, 16:32] = a1
        return carry

    lax.fori_loop(0, _L, l_body, 0)
    pltpu.sync_copy(res_v, out_hbm.at[wid])


_gather_sum = functools.partial(
    pl.kernel,
    out_type=jax.ShapeDtypeStruct((_NW, _L, _DP), jnp.float32),
    mesh=plsc.VectorSubcoreMesh(core_axis_name="c", subcore_axis_name="s"),
    compiler_params=pltpu.CompilerParams(use_tc_tiling_on_sc=False),
    scratch_types=[
        pltpu.VMEM((_L, _BW), jnp.int32),
        pltpu.VMEM((_NCH, _CH, _DP), jnp.float32),
        pltpu.VMEM((_L, _DP), jnp.float32),
        pltpu.SemaphoreType.DMA,
        pltpu.SemaphoreType.DMA,
        pltpu.SemaphoreType.DMA,
        pltpu.SemaphoreType.DMA,
    ],
)(_gather_body)


def kernel(inputs, tables):
    planes = [tables[t, :, c] for t in range(_NT) for c in range(_E)]
    tabp = _relayout(*planes)
    idx_t = inputs.T
    parts = _gather_sum(idx_t, tabp)
    m = jnp.sum(parts, axis=0)[:, :_D].reshape(_L, _NT, _E)
    s3 = jnp.sum(m[:, 3, :], axis=0, keepdims=True)
    return jnp.concatenate(
        [m[:, 0], m[:, 1], m[:, 2], s3, m[:, 4], s3,
         m[:, 6], m[:, 7], m[:, 8], m[:, 9]],
        axis=0,
    )
